# baseline (speedup 1.0000x reference)
import functools

import jax
import jax.numpy as jnp
from jax.experimental import pallas as pl
import jax.experimental.pallas.tpu as pltpu


def _blk_kernel(idx_ref, w_ref, x_ref, b_ref, out_ref, s0_ref, s1_ref, *,
                in_features, bo, n_blk):
    fan = idx_ref.shape[0]
    j = pl.program_id(0)
    bufs = [s0_ref, s1_ref]

    @pl.when(j < n_blk)
    def _build():
        iota = jax.lax.broadcasted_iota(jnp.int16, (in_features, bo), 0)
        idx = idx_ref[...]
        w = w_ref[...]
        s = jnp.zeros((in_features, bo), jnp.bfloat16)
        for f in range(fan):
            s = jnp.where(iota == idx[f : f + 1, :], w[f : f + 1, :], s)
        sf = s.astype(jnp.float32)

        @pl.when(j % 2 == 0)
        def _even():
            bufs[0][...] = sf

        @pl.when(j % 2 == 1)
        def _odd():
            bufs[1][...] = sf

    @pl.when(j > 0)
    def _dot():
        @pl.when(j % 2 == 1)
        def _use_even():
            out_ref[...] = (
                jnp.dot(
                    x_ref[...], bufs[0][...],
                    preferred_element_type=jnp.float32,
                )
                + b_ref[...]
            )

        @pl.when(j % 2 == 0)
        def _use_odd():
            out_ref[...] = (
                jnp.dot(
                    x_ref[...], bufs[1][...],
                    preferred_element_type=jnp.float32,
                )
                + b_ref[...]
            )


def kernel(input, weight, bias, indx_seqs):
    batch, in_features = input.shape
    out_features, fan_in = weight.shape
    bo = min(256, out_features)
    n_blk = out_features // bo

    idx_t = indx_seqs.astype(jnp.int16).T
    w_t = weight.T.astype(jnp.bfloat16)
    bias2 = bias.reshape(1, out_features)

    out = pl.pallas_call(
        functools.partial(
            _blk_kernel, in_features=in_features, bo=bo, n_blk=n_blk
        ),
        grid=(n_blk + 1,),
        in_specs=[
            pl.BlockSpec((fan_in, bo), lambda j: (0, jnp.minimum(j, n_blk - 1))),
            pl.BlockSpec((fan_in, bo), lambda j: (0, jnp.minimum(j, n_blk - 1))),
            pl.BlockSpec((batch, in_features), lambda j: (0, 0)),
            pl.BlockSpec((1, bo), lambda j: (0, jnp.maximum(j - 1, 0))),
        ],
        out_specs=pl.BlockSpec((batch, bo), lambda j: (0, jnp.maximum(j - 1, 0))),
        out_shape=jax.ShapeDtypeStruct((batch, out_features), jnp.float32),
        scratch_shapes=[
            pltpu.VMEM((in_features, bo), jnp.float32),
            pltpu.VMEM((in_features, bo), jnp.float32),
        ],
    )(idx_t, w_t, input, bias2)
    return out


# speedup vs baseline: 1.1648x; 1.1648x over previous
import functools

import jax
import jax.numpy as jnp
from jax.experimental import pallas as pl
import jax.experimental.pallas.tpu as pltpu


def _blk_kernel(idx_ref, w_ref, x_ref, b_ref, out_ref, s0_ref, *,
                in_features, bo, n_blk):
    fan = idx_ref.shape[0]
    j = pl.program_id(0)

    def _build_s():
        iota = jax.lax.broadcasted_iota(jnp.int16, (in_features, bo), 0)
        idx = idx_ref[...]
        w = w_ref[...]
        s = jnp.zeros((in_features, bo), jnp.bfloat16)
        for f in range(fan):
            s = jnp.where(iota == idx[f : f + 1, :], w[f : f + 1, :], s)
        return s.astype(jnp.float32)

    @pl.when(j == 0)
    def _prologue():
        s0_ref[...] = _build_s()

    @pl.when(j > 0)
    def _dot_and_build_next():
        out_ref[...] = (
            jnp.dot(
                x_ref[...], s0_ref[...], preferred_element_type=jnp.float32
            )
            + b_ref[...]
        )
        s0_ref[...] = _build_s()


def kernel(input, weight, bias, indx_seqs):
    batch, in_features = input.shape
    out_features, fan_in = weight.shape
    bo = min(256, out_features)
    n_blk = out_features // bo

    idx_t = indx_seqs.astype(jnp.int16).T
    w_t = weight.T.astype(jnp.bfloat16)
    bias2 = bias.reshape(1, out_features)

    out = pl.pallas_call(
        functools.partial(
            _blk_kernel, in_features=in_features, bo=bo, n_blk=n_blk
        ),
        grid=(n_blk + 1,),
        in_specs=[
            pl.BlockSpec((fan_in, bo), lambda j: (0, jnp.minimum(j, n_blk - 1))),
            pl.BlockSpec((fan_in, bo), lambda j: (0, jnp.minimum(j, n_blk - 1))),
            pl.BlockSpec((batch, in_features), lambda j: (0, 0)),
            pl.BlockSpec((1, bo), lambda j: (0, jnp.maximum(j - 1, 0))),
        ],
        out_specs=pl.BlockSpec((batch, bo), lambda j: (0, jnp.maximum(j - 1, 0))),
        out_shape=jax.ShapeDtypeStruct((batch, out_features), jnp.float32),
        scratch_shapes=[
            pltpu.VMEM((in_features, bo), jnp.float32),
        ],
    )(idx_t, w_t, input, bias2)
    return out


# speedup vs baseline: 1.2106x; 1.0393x over previous
import functools

import jax
import jax.numpy as jnp
from jax.experimental import pallas as pl


def _blk_kernel(idx_ref, w_ref, x_ref, b_ref, out_ref, *, in_features, bo):
    fan = idx_ref.shape[0]
    iota = jax.lax.broadcasted_iota(jnp.int16, (in_features, bo), 0)
    idx = idx_ref[...]
    w = w_ref[...]
    s = jnp.zeros((in_features, bo), jnp.bfloat16)
    for f in range(fan):
        s = jnp.where(iota == idx[f : f + 1, :], w[f : f + 1, :], s)
    out_ref[...] = (
        jnp.dot(
            x_ref[...],
            s.astype(jnp.float32),
            preferred_element_type=jnp.float32,
        )
        + b_ref[...]
    )


def kernel(input, weight, bias, indx_seqs):
    batch, in_features = input.shape
    out_features, fan_in = weight.shape
    bo = min(512, out_features)
    n_blk = out_features // bo

    idx_t = indx_seqs.astype(jnp.int16).T
    w_t = weight.T.astype(jnp.bfloat16)
    bias2 = bias.reshape(1, out_features)

    out = pl.pallas_call(
        functools.partial(_blk_kernel, in_features=in_features, bo=bo),
        grid=(n_blk,),
        in_specs=[
            pl.BlockSpec((fan_in, bo), lambda j: (0, j)),
            pl.BlockSpec((fan_in, bo), lambda j: (0, j)),
            pl.BlockSpec((batch, in_features), lambda j: (0, 0)),
            pl.BlockSpec((1, bo), lambda j: (0, j)),
        ],
        out_specs=pl.BlockSpec((batch, bo), lambda j: (0, j)),
        out_shape=jax.ShapeDtypeStruct((batch, out_features), jnp.float32),
    )(idx_t, w_t, input, bias2)
    return out


# speedup vs baseline: 1.2246x; 1.0115x over previous
import functools

import jax
import jax.numpy as jnp
from jax.experimental import pallas as pl


def _blk_kernel(idx_ref, w_ref, x_ref, b_ref, out_ref, *, in_features, bo):
    fan = idx_ref.shape[0]
    iota = jax.lax.broadcasted_iota(jnp.int16, (in_features, bo), 0)
    idx = idx_ref[...]
    w = w_ref[...]
    s = jnp.zeros((in_features, bo), jnp.bfloat16)
    for f in range(fan):
        s = jnp.where(iota == idx[f : f + 1, :], w[f : f + 1, :], s)
    out_ref[...] = (
        jnp.dot(
            x_ref[...],
            s.astype(jnp.float32),
            preferred_element_type=jnp.float32,
        )
        + b_ref[...]
    )


def kernel(input, weight, bias, indx_seqs):
    batch, in_features = input.shape
    out_features, fan_in = weight.shape
    bo = min(256, out_features)
    n_blk = out_features // bo

    idx_t = indx_seqs.astype(jnp.int16).T
    w_t = weight.T.astype(jnp.bfloat16)
    bias2 = bias.reshape(1, out_features)

    out = pl.pallas_call(
        functools.partial(_blk_kernel, in_features=in_features, bo=bo),
        grid=(n_blk,),
        in_specs=[
            pl.BlockSpec((fan_in, bo), lambda j: (0, j)),
            pl.BlockSpec((fan_in, bo), lambda j: (0, j)),
            pl.BlockSpec((batch, in_features), lambda j: (0, 0)),
            pl.BlockSpec((1, bo), lambda j: (0, j)),
        ],
        out_specs=pl.BlockSpec((batch, bo), lambda j: (0, j)),
        out_shape=jax.ShapeDtypeStruct((batch, out_features), jnp.float32),
    )(idx_t, w_t, input, bias2)
    return out
